# baseline (speedup 1.0000x reference)
import functools
import math

import jax
import jax.numpy as jnp
import numpy as np
from jax.experimental import pallas as pl
from jax.experimental.pallas import tpu as pltpu

_IMAGE_SIZE = 1024.0
_C = 256
_NPF = _C // 2
_K = 4


def _coeff_table():
    i = np.arange(_NPF, dtype=np.float64)
    dim_t = 10000.0 ** (2.0 * np.floor(i / 2.0) / _NPF)
    freq = (2.0 * math.pi / _IMAGE_SIZE) / dim_t
    even = (np.arange(_NPF) % 2) == 0
    k = np.zeros((6, _NPF), dtype=np.float64)
    k[1, even], k[3, even], k[5, even] = 1.0, -1.0 / 6.0, 1.0 / 120.0
    k[0, ~even], k[2, ~even], k[4, ~even] = 1.0, -0.5, 1.0 / 24.0
    out = np.zeros((8, _NPF), dtype=np.float64)
    out[0] = freq
    out[1:7] = k
    return out.astype(np.float32)


_COEFFS = _coeff_table()


def _body(pts_ref, bxs_ref, feats_ref, coef_ref, rows_ref, out1_ref, out2_ref,
          buf_ref, semv_ref, semh_ref, *, nq, tq, nsteps):
    b = pl.program_id(0)
    q = pl.program_id(1)
    step = b * nq + q
    slot = jax.lax.rem(step, _K)
    c = feats_ref.shape[2]

    def v_copy(s, bb, qq):
        return pltpu.make_async_copy(
            buf_ref.at[s],
            out1_ref.at[:, bb, pl.ds(qq * tq, tq)],
            semv_ref.at[s])

    def h_copy(s, bb, qq):
        return pltpu.make_async_copy(
            out1_ref.at[:, bb, pl.ds(qq * tq, tq)],
            out2_ref.at[:, bb, pl.ds(qq * tq, tq)],
            semh_ref.at[s])

    @pl.when(step >= 2 * _K)
    def _wait_h():
        h_copy(slot, b, q).wait()

    @pl.when(step >= _K)
    def _chain():
        v_copy(slot, b, q).wait()
        bj = jax.lax.div(step - _K, nq)
        qj = jax.lax.rem(step - _K, nq)
        h_copy(slot, bj, qj).start()

    freq = coef_ref[0, :]
    k0, k1, k2 = coef_ref[1, :], coef_ref[2, :], coef_ref[3, :]
    k3, k4, k5 = coef_ref[4, :], coef_ref[5, :], coef_ref[6, :]
    content = feats_ref[0]

    def pe_half(coord):
        ang = coord * freq[None, :]
        a2 = ang * ang
        even_p = k0[None, :] + a2 * (k2[None, :] + a2 * k4[None, :])
        odd_p = k1[None, :] + a2 * (k3[None, :] + a2 * k5[None, :])
        return even_p + ang * odd_p

    def pe(x, y):
        return jnp.concatenate([pe_half(y), pe_half(x)], axis=1)

    x = pts_ref[0, :, 0:1]
    y = pts_ref[0, :, 1:2]
    buf_ref[slot, 0, :, 0, :] = pe(x, y) + (content + rows_ref[0, :][None, :])
    buf_ref[slot, 0, :, 1, :] = jnp.broadcast_to(rows_ref[3, :], (tq, c))
    buf_ref[slot, 0, :, 2, :] = jnp.broadcast_to(rows_ref[4, :], (tq, c))

    x1 = bxs_ref[0, :, 0:1]
    y1 = bxs_ref[0, :, 1:2]
    x2 = bxs_ref[0, :, 2:3]
    y2 = bxs_ref[0, :, 3:4]
    buf_ref[slot, 1, :, 0, :] = pe(x1, y1) + (content + rows_ref[1, :][None, :])
    buf_ref[slot, 1, :, 1, :] = pe(x2, y2) + (content + rows_ref[2, :][None, :])
    buf_ref[slot, 1, :, 2, :] = jnp.broadcast_to(rows_ref[5, :], (tq, c))

    v_copy(slot, b, q).start()

    @pl.when(step == nsteps - 1)
    def _drain():
        for r in range(nsteps - 2 * _K, nsteps - _K):
            h_copy(r % _K, b, q).wait()
        for r in range(nsteps - _K, nsteps):
            v_copy(r % _K, b, q).wait()
        for r in range(nsteps - _K, nsteps):
            h_copy(r % _K, r // nq, r % nq).start()
        for r in range(nsteps - _K, nsteps):
            h_copy(r % _K, b, q).wait()


def kernel(points, boxes, points_multi, feats_centers, corner_emb, point_emb,
           box_emb, attr_emb_weight, mask_emb):
    del points_multi
    B, Q, C = feats_centers.shape
    TQ = 512
    NQ = Q // TQ

    rowconsts = jnp.stack([
        point_emb[0, 0] + attr_emb_weight[1],
        corner_emb[0, 0] + box_emb[0, 0],
        corner_emb[0, 1] + box_emb[0, 0],
        mask_emb[0, -2],
        mask_emb[0, -1],
        mask_emb[0, 0],
        jnp.zeros((C,), jnp.float32),
        jnp.zeros((C,), jnp.float32),
    ])
    coeffs = jnp.asarray(_COEFFS)

    grid = (B, NQ)
    out_sds = jax.ShapeDtypeStruct((2, B, Q, 3, C), jnp.float32)
    out1, out2 = pl.pallas_call(
        functools.partial(_body, nq=NQ, tq=TQ, nsteps=B * NQ),
        grid=grid,
        in_specs=[
            pl.BlockSpec((1, TQ, 2), lambda b, q: (b, q, 0)),
            pl.BlockSpec((1, TQ, 4), lambda b, q: (b, q, 0)),
            pl.BlockSpec((1, TQ, C), lambda b, q: (b, q, 0)),
            pl.BlockSpec((8, C // 2), lambda b, q: (0, 0)),
            pl.BlockSpec((8, C), lambda b, q: (0, 0)),
        ],
        out_specs=[
            pl.BlockSpec(memory_space=pltpu.MemorySpace.HBM),
            pl.BlockSpec(memory_space=pltpu.MemorySpace.HBM),
        ],
        out_shape=[out_sds, out_sds],
        scratch_shapes=[
            pltpu.VMEM((_K, 2, TQ, 3, C), jnp.float32),
            pltpu.SemaphoreType.DMA((_K,)),
            pltpu.SemaphoreType.DMA((_K,)),
        ],
        compiler_params=pltpu.CompilerParams(
            dimension_semantics=("arbitrary", "arbitrary"),
        ),
    )(points, boxes, feats_centers, coeffs, rowconsts)
    shape = (2 * B, Q, 3, C)
    return (out1.reshape(shape), out2.reshape(shape))


# speedup vs baseline: 18.6070x; 18.6070x over previous
import math

import jax
import jax.numpy as jnp
import numpy as np
from jax.experimental import pallas as pl
from jax.experimental.pallas import tpu as pltpu

_IMAGE_SIZE = 1024.0
_C = 256
_NPF = _C // 2


def _coeff_table():
    i = np.arange(_NPF, dtype=np.float64)
    dim_t = 10000.0 ** (2.0 * np.floor(i / 2.0) / _NPF)
    freq = (2.0 * math.pi / _IMAGE_SIZE) / dim_t
    even = (np.arange(_NPF) % 2) == 0
    k = np.zeros((6, _NPF), dtype=np.float64)
    k[1, even], k[3, even], k[5, even] = 1.0, -1.0 / 6.0, 1.0 / 120.0
    k[0, ~even], k[2, ~even], k[4, ~even] = 1.0, -0.5, 1.0 / 24.0
    out = np.zeros((8, _NPF), dtype=np.float64)
    out[0] = freq
    out[1:7] = k
    return out.astype(np.float32)


_COEFFS = _coeff_table()


def _body(pts_ref, bxs_ref, feats_ref, coef_ref, rows_ref, out_ref):
    tq = feats_ref.shape[1]
    c = feats_ref.shape[2]
    freq = coef_ref[0, :]
    k0, k1, k2 = coef_ref[1, :], coef_ref[2, :], coef_ref[3, :]
    k3, k4, k5 = coef_ref[4, :], coef_ref[5, :], coef_ref[6, :]
    content = feats_ref[0]

    def pe_half(coord):
        ang = coord * freq[None, :]
        a2 = ang * ang
        even_p = k0[None, :] + a2 * (k2[None, :] + a2 * k4[None, :])
        odd_p = k1[None, :] + a2 * (k3[None, :] + a2 * k5[None, :])
        return even_p + ang * odd_p

    def pe(x, y):
        return jnp.concatenate([pe_half(y), pe_half(x)], axis=1)

    x = pts_ref[0, :, 0:1]
    y = pts_ref[0, :, 1:2]
    out_ref[0, 0, :, 0, :] = pe(x, y) + (content + rows_ref[0, :][None, :])
    out_ref[0, 0, :, 1, :] = jnp.broadcast_to(rows_ref[3, :], (tq, c))
    out_ref[0, 0, :, 2, :] = jnp.broadcast_to(rows_ref[4, :], (tq, c))

    x1 = bxs_ref[0, :, 0:1]
    y1 = bxs_ref[0, :, 1:2]
    x2 = bxs_ref[0, :, 2:3]
    y2 = bxs_ref[0, :, 3:4]
    out_ref[1, 0, :, 0, :] = pe(x1, y1) + (content + rows_ref[1, :][None, :])
    out_ref[1, 0, :, 1, :] = pe(x2, y2) + (content + rows_ref[2, :][None, :])
    out_ref[1, 0, :, 2, :] = jnp.broadcast_to(rows_ref[5, :], (tq, c))


def kernel(points, boxes, points_multi, feats_centers, corner_emb, point_emb,
           box_emb, attr_emb_weight, mask_emb):
    del points_multi
    B, Q, C = feats_centers.shape
    TQ = 1024

    rowconsts = jnp.stack([
        point_emb[0, 0] + attr_emb_weight[1],
        corner_emb[0, 0] + box_emb[0, 0],
        corner_emb[0, 1] + box_emb[0, 0],
        mask_emb[0, -2],
        mask_emb[0, -1],
        mask_emb[0, 0],
        jnp.zeros((C,), jnp.float32),
        jnp.zeros((C,), jnp.float32),
    ])
    coeffs = jnp.asarray(_COEFFS)

    grid = (B, Q // TQ)
    out = pl.pallas_call(
        _body,
        grid=grid,
        in_specs=[
            pl.BlockSpec((1, TQ, 2), lambda b, q: (b, q, 0)),
            pl.BlockSpec((1, TQ, 4), lambda b, q: (b, q, 0)),
            pl.BlockSpec((1, TQ, C), lambda b, q: (b, q, 0)),
            pl.BlockSpec((8, C // 2), lambda b, q: (0, 0)),
            pl.BlockSpec((8, C), lambda b, q: (0, 0)),
        ],
        out_specs=pl.BlockSpec((2, 1, TQ, 3, C), lambda b, q: (0, b, q, 0, 0)),
        out_shape=jax.ShapeDtypeStruct((2, B, Q, 3, C), jnp.float32),
        compiler_params=pltpu.CompilerParams(
            dimension_semantics=("parallel", "parallel"),
        ),
    )(points, boxes, feats_centers, coeffs, rowconsts)
    out = out.reshape(2 * B, Q, 3, C)
    return (out, out)
